# baseline (speedup 1.0000x reference)
import functools

import jax
import jax.numpy as jnp
from jax import lax
from jax.experimental import pallas as pl
from jax.experimental.pallas import tpu as pltpu
from jax.experimental.pallas import tpu_sc as plsc

NC = 2
NS = 16
LANES = 16
C = 80


def _preproject(x, w1a, w1b):
    n, d = x.shape
    nb = 1000

    def body(x_ref, wa_ref, wb_ref, pa_ref, pb_ref):
        xv = x_ref[...]
        pa_ref[...] = jnp.dot(xv, wa_ref[...], preferred_element_type=jnp.float32)
        pb_ref[...] = jnp.dot(xv, wb_ref[...], preferred_element_type=jnp.float32)

    return pl.pallas_call(
        body,
        grid=(n // nb,),
        in_specs=[
            pl.BlockSpec((nb, d), lambda i: (i, 0)),
            pl.BlockSpec((d, d), lambda i: (0, 0)),
            pl.BlockSpec((d, d), lambda i: (0, 0)),
        ],
        out_specs=[pl.BlockSpec((nb, d), lambda i: (i, 0))] * 2,
        out_shape=[jax.ShapeDtypeStruct((n, d), jnp.float32)] * 2,
    )(x, w1a, w1b)


def _sc_gather_add(pa, pb, row3d, col3d, e):
    n, d = pa.shape
    nw = NC * NS
    epw = e // nw
    ch = epw // C

    mesh = plsc.VectorSubcoreMesh(core_axis_name="c", subcore_axis_name="s")

    def body(pa_hbm, pb_hbm, row_hbm, col_hbm, out_hbm,
             rowv, colv, ba0, bb0, ba1, bb1, sb0_, sb1_,
             sa0, sbm0, so0, sa1, sbm1, so1):
        cid = lax.axis_index("c")
        sid = lax.axis_index("s")
        wid = sid * NC + cid
        pltpu.sync_copy(row_hbm.at[wid], rowv)
        pltpu.sync_copy(col_hbm.at[wid], colv)

        bufs = ((ba0, bb0, sb0_, sa0, sbm0, so0),
                (ba1, bb1, sb1_, sa1, sbm1, so1))

        def out_slice(jj):
            return out_hbm.at[pl.ds(
                pl.multiple_of(wid * epw + jj * C, 16), C)]

        def issue(jj, p):
            ba, bb, _, sa, sb, _ = bufs[p]
            pltpu.async_copy(pa_hbm.at[rowv.at[jj]], ba, sa)
            pltpu.async_copy(pb_hbm.at[colv.at[jj]], bb, sb)

        def wait_store(jj, p):
            sbuf = bufs[p][2]
            so = bufs[p][5]
            pltpu.make_async_copy(sbuf, out_slice(jj), so).wait()

        issue(0, 0)

        def chunk(j, carry):
            for p in (0, 1):
                @pl.when(j % 2 == p)
                def _():
                    @pl.when(j + 1 < ch)
                    def _():
                        issue(j + 1, 1 - p)
                    pltpu.make_async_copy(
                        pa_hbm.at[rowv.at[j]], bufs[p][0], bufs[p][3]).wait()
                    pltpu.make_async_copy(
                        pb_hbm.at[colv.at[j]], bufs[p][1], bufs[p][4]).wait()
                    @pl.when(j >= 2)
                    def _():
                        wait_store(j - 2, p)

                    def addrow(r, c2):
                        sbuf = bufs[p][2]
                        for k in range(d // LANES):
                            s16 = pl.ds(k * LANES, LANES)
                            sbuf[r, s16] = bufs[p][0][r, s16] + bufs[p][1][r, s16]
                        return c2

                    lax.fori_loop(0, C, addrow, 0)
                    pltpu.async_copy(bufs[p][2], out_slice(j), bufs[p][5])
            return carry

        lax.fori_loop(0, ch, chunk, 0)
        wait_store(ch - 1, (ch - 1) % 2)
        wait_store(ch - 2, (ch - 2) % 2)

    f = pl.kernel(
        body,
        out_type=jax.ShapeDtypeStruct((e, d), jnp.float32),
        mesh=mesh,
        scratch_types=[
            pltpu.VMEM((ch, C), jnp.int32),
            pltpu.VMEM((ch, C), jnp.int32),
            pltpu.VMEM((C, d), jnp.float32),
            pltpu.VMEM((C, d), jnp.float32),
            pltpu.VMEM((C, d), jnp.float32),
            pltpu.VMEM((C, d), jnp.float32),
            pltpu.VMEM((C, d), jnp.float32),
            pltpu.VMEM((C, d), jnp.float32),
            pltpu.SemaphoreType.DMA,
            pltpu.SemaphoreType.DMA,
            pltpu.SemaphoreType.DMA,
            pltpu.SemaphoreType.DMA,
            pltpu.SemaphoreType.DMA,
            pltpu.SemaphoreType.DMA,
        ],
    )
    return f(pa, pb, row3d, col3d)


def _edge_tail_slice(g, edge_attr, mask, w1c, b1, w2_bf, b2, slice_idx):
    es, d = g.shape
    de = edge_attr.shape[1]
    eb = 1600
    nblk = es // eb
    off = slice_idx * nblk

    def body(g_ref, ea_ref, m_ref, w1c_ref, b1_ref, w2_ref, b2_ref, ef_ref):
        pre = (g_ref[...]
               + jnp.dot(ea_ref[...], w1c_ref[...],
                         preferred_element_type=jnp.float32)
               + b1_ref[...])
        h = jnp.maximum(pre, 0.0).astype(jnp.bfloat16)
        ef_ref[...] = (jnp.dot(h, w2_ref[...],
                               preferred_element_type=jnp.float32)
                       + b2_ref[...]) * m_ref[...]

    return pl.pallas_call(
        body,
        grid=(nblk,),
        in_specs=[
            pl.BlockSpec((eb, d), lambda i: (i, 0)),
            pl.BlockSpec((eb, de), lambda i: (off + i, 0)),
            pl.BlockSpec((eb, 1), lambda i: (off + i, 0)),
            pl.BlockSpec((de, d), lambda i: (0, 0)),
            pl.BlockSpec((1, d), lambda i: (0, 0)),
            pl.BlockSpec((d, d), lambda i: (0, 0)),
            pl.BlockSpec((1, d), lambda i: (0, 0)),
        ],
        out_specs=pl.BlockSpec((eb, d), lambda i: (i, 0)),
        out_shape=jax.ShapeDtypeStruct((es, d), jnp.float32),
    )(g, edge_attr, mask, w1c, b1, w2_bf, b2)


def _sc_scatter_add(efs, row3d, n):
    ns = len(efs)
    es, d = efs[0].shape
    e = es * ns
    nw = NC * NS
    epw = es // nw
    ch = epw // C
    tr = 80
    nt = n // tr
    tpw = (nt + NS - 1) // NS

    mesh = plsc.VectorSubcoreMesh(core_axis_name="c", subcore_axis_name="s")

    def body(*refs):
        ef_hbms = refs[:ns]
        row_hbm = refs[ns]
        out_hbm = refs[ns + 1]
        acc, rowv, eb0, eb1, zbuf, se0, se1 = refs[ns + 2:]
        cid = lax.axis_index("c")
        sid = lax.axis_index("s")
        wid = sid * NC + cid

        def zrow(r, c2):
            for k in range(d // LANES):
                zbuf[r, pl.ds(k * LANES, LANES)] = jnp.zeros((LANES,), jnp.float32)
            return c2

        lax.fori_loop(0, tr, zrow, 0)
        for t in range(tpw):
            tile = sid + t * NS
            @pl.when(tile < nt)
            def _():
                pltpu.sync_copy(zbuf, acc.at[pl.ds(tile * tr, tr)])
        plsc.subcore_barrier()

        pltpu.sync_copy(row_hbm.at[wid], rowv)

        bufs = ((eb0, se0), (eb1, se1))

        for si in range(ns):
            ef_hbm = ef_hbms[si]

            def issue(jj, p):
                buf, sem = bufs[p]
                pltpu.async_copy(
                    ef_hbm.at[pl.ds(wid * epw + jj * C, C)], buf, sem)

            issue(0, 0)

            def chunk(j, carry):
                for p in (0, 1):
                    @pl.when(j % 2 == p)
                    def _():
                        buf, sem = bufs[p]

                        @pl.when(j + 1 < ch)
                        def _():
                            issue(j + 1, 1 - p)
                        pltpu.make_async_copy(
                            ef_hbm.at[pl.ds(wid * epw + j * C, C)],
                            buf, sem).wait()
                        pltpu.sync_copy(buf, acc.at[rowv.at[si * ch + j]],
                                        add=True)
                return carry

            lax.fori_loop(0, ch, chunk, 0)

        plsc.subcore_barrier()
        for t in range(tpw):
            tile = sid + t * NS
            @pl.when(tile < nt)
            def _():
                pltpu.sync_copy(acc.at[pl.ds(tile * tr, tr)],
                                out_hbm.at[cid, pl.ds(tile * tr, tr)])

    f = pl.kernel(
        body,
        out_type=jax.ShapeDtypeStruct((NC, n, d), jnp.float32),
        mesh=mesh,
        scratch_types=[
            pltpu.VMEM_SHARED((n, d), jnp.float32),
            pltpu.VMEM((ns * ch, C), jnp.int32),
            pltpu.VMEM((C, d), jnp.float32),
            pltpu.VMEM((C, d), jnp.float32),
            pltpu.VMEM((tr, d), jnp.float32),
            pltpu.SemaphoreType.DMA,
            pltpu.SemaphoreType.DMA,
        ],
    )
    return f(*efs, row3d)


def _node_mlp(x, aggp, wn1a, wn1b, bn1, wn2, bn2):
    n, d = x.shape
    nb = 1000

    def body(x_ref, a0_ref, a1_ref, wa_ref, wb_ref, b1_ref, w2_ref, b2_ref,
             out_ref):
        xv = x_ref[...]
        agg = a0_ref[0] + a1_ref[0]
        hn = jnp.maximum(
            jnp.dot(xv, wa_ref[...], preferred_element_type=jnp.float32)
            + jnp.dot(agg, wb_ref[...], preferred_element_type=jnp.float32)
            + b1_ref[...], 0.0)
        out_ref[...] = (xv
                        + jnp.dot(hn, w2_ref[...],
                                  preferred_element_type=jnp.float32)
                        + b2_ref[...])

    return pl.pallas_call(
        body,
        grid=(n // nb,),
        in_specs=[
            pl.BlockSpec((nb, d), lambda i: (i, 0)),
            pl.BlockSpec((1, nb, d), lambda i: (0, i, 0)),
            pl.BlockSpec((1, nb, d), lambda i: (1, i, 0)),
            pl.BlockSpec((d, d), lambda i: (0, 0)),
            pl.BlockSpec((d, d), lambda i: (0, 0)),
            pl.BlockSpec((1, d), lambda i: (0, 0)),
            pl.BlockSpec((d, d), lambda i: (0, 0)),
            pl.BlockSpec((1, d), lambda i: (0, 0)),
        ],
        out_specs=pl.BlockSpec((nb, d), lambda i: (i, 0)),
        out_shape=jax.ShapeDtypeStruct((n, d), jnp.float32),
    )(x, aggp, aggp, wn1a, wn1b, bn1, wn2, bn2)


def kernel(x, edge_index, edge_mask, edge_attr, W1, b1, W2, b2,
           Wn1, bn1, Wn2, bn2):
    n, d = x.shape
    e = edge_index.shape[1]
    nw = NC * NS
    ns = 5
    es = e // ns
    row4d = edge_index[0].reshape(ns, nw, es // (nw * C), C)
    col4d = edge_index[1].reshape(ns, nw, es // (nw * C), C)
    row_sc = row4d.transpose(1, 0, 2, 3).reshape(nw, e // (nw * C), C)

    pa, pb = _preproject(x, W1[:d], W1[d:2 * d])
    gs = [_sc_gather_add(pa, pb, row4d[i], col4d[i], es) for i in range(ns)]
    w1c = W1[2 * d:]
    w2_bf = W2.astype(jnp.bfloat16)
    b1r = b1.reshape(1, d)
    b2r = b2.reshape(1, d)
    efs = [_edge_tail_slice(gs[i], edge_attr, edge_mask, w1c,
                            b1r, w2_bf, b2r, i) for i in range(ns)]
    aggp = _sc_scatter_add(efs, row_sc, n)
    ef = jnp.concatenate(efs, axis=0)
    x_out = _node_mlp(x, aggp, Wn1[:d], Wn1[d:], bn1.reshape(1, d),
                      Wn2, bn2.reshape(1, d))
    return x_out, ef


# speedup vs baseline: 1.3045x; 1.3045x over previous
import functools

import jax
import jax.numpy as jnp
from jax import lax
from jax.experimental import pallas as pl
from jax.experimental.pallas import tpu as pltpu
from jax.experimental.pallas import tpu_sc as plsc

NC = 2
NS = 16
LANES = 16
C = 80


def _preproject(x, w1a, w1b):
    n, d = x.shape
    nb = 1000

    def body(x_ref, wa_ref, wb_ref, pa_ref, pb_ref):
        xv = x_ref[...]
        pa_ref[...] = jnp.dot(xv, wa_ref[...], preferred_element_type=jnp.float32)
        pb_ref[...] = jnp.dot(xv, wb_ref[...], preferred_element_type=jnp.float32)

    return pl.pallas_call(
        body,
        grid=(n // nb,),
        in_specs=[
            pl.BlockSpec((nb, d), lambda i: (i, 0)),
            pl.BlockSpec((d, d), lambda i: (0, 0)),
            pl.BlockSpec((d, d), lambda i: (0, 0)),
        ],
        out_specs=[pl.BlockSpec((nb, d), lambda i: (i, 0))] * 2,
        out_shape=[jax.ShapeDtypeStruct((n, d), jnp.float32)] * 2,
    )(x, w1a, w1b)


def _sc_gather_add(pa, pb, row3d, col3d, e):
    n, d = pa.shape
    nw = NC * NS
    epw = e // nw
    ch = epw // C

    mesh = plsc.VectorSubcoreMesh(core_axis_name="c", subcore_axis_name="s")

    def body(pa_hbm, pb_hbm, row_hbm, col_hbm, out_hbm,
             rowv, colv, ba0, bb0, ba1, bb1, sb0_, sb1_,
             sa0, sbm0, so0, sa1, sbm1, so1):
        cid = lax.axis_index("c")
        sid = lax.axis_index("s")
        wid = sid * NC + cid
        pltpu.sync_copy(row_hbm.at[wid], rowv)
        pltpu.sync_copy(col_hbm.at[wid], colv)

        bufs = ((ba0, bb0, sb0_, sa0, sbm0, so0),
                (ba1, bb1, sb1_, sa1, sbm1, so1))

        def out_slice(jj):
            return out_hbm.at[pl.ds(
                pl.multiple_of(wid * epw + jj * C, 16), C)]

        def issue(jj, p):
            ba, bb, _, sa, sb, _ = bufs[p]
            pltpu.async_copy(pa_hbm.at[rowv.at[jj]], ba, sa)
            pltpu.async_copy(pb_hbm.at[colv.at[jj]], bb, sb)

        def wait_store(jj, p):
            sbuf = bufs[p][2]
            so = bufs[p][5]
            pltpu.make_async_copy(sbuf, out_slice(jj), so).wait()

        issue(0, 0)

        def chunk(j, carry):
            for p in (0, 1):
                @pl.when(j % 2 == p)
                def _():
                    @pl.when(j + 1 < ch)
                    def _():
                        issue(j + 1, 1 - p)
                    pltpu.make_async_copy(
                        pa_hbm.at[rowv.at[j]], bufs[p][0], bufs[p][3]).wait()
                    pltpu.make_async_copy(
                        pb_hbm.at[colv.at[j]], bufs[p][1], bufs[p][4]).wait()
                    @pl.when(j >= 2)
                    def _():
                        wait_store(j - 2, p)

                    def addrow(r, c2):
                        sbuf = bufs[p][2]
                        for k in range(d // LANES):
                            s16 = pl.ds(k * LANES, LANES)
                            sbuf[r, s16] = bufs[p][0][r, s16] + bufs[p][1][r, s16]
                        return c2

                    lax.fori_loop(0, C, addrow, 0)
                    pltpu.async_copy(bufs[p][2], out_slice(j), bufs[p][5])
            return carry

        lax.fori_loop(0, ch, chunk, 0)
        wait_store(ch - 1, (ch - 1) % 2)
        wait_store(ch - 2, (ch - 2) % 2)

    f = pl.kernel(
        body,
        out_type=jax.ShapeDtypeStruct((e, d), jnp.float32),
        mesh=mesh,
        scratch_types=[
            pltpu.VMEM((ch, C), jnp.int32),
            pltpu.VMEM((ch, C), jnp.int32),
            pltpu.VMEM((C, d), jnp.float32),
            pltpu.VMEM((C, d), jnp.float32),
            pltpu.VMEM((C, d), jnp.float32),
            pltpu.VMEM((C, d), jnp.float32),
            pltpu.VMEM((C, d), jnp.float32),
            pltpu.VMEM((C, d), jnp.float32),
            pltpu.SemaphoreType.DMA,
            pltpu.SemaphoreType.DMA,
            pltpu.SemaphoreType.DMA,
            pltpu.SemaphoreType.DMA,
            pltpu.SemaphoreType.DMA,
            pltpu.SemaphoreType.DMA,
        ],
    )
    return f(pa, pb, row3d, col3d)


def _edge_tail_slice(g, ea_p, mask_p, w1c, b1, w2_bf, b2, slice_idx):
    es, d = g.shape
    de = 16
    eb = 1280
    nblk = es // eb
    off = slice_idx * nblk

    def body(g_ref, ea_ref, m_ref, w1c_ref, b1_ref, w2_ref, b2_ref, ef_ref):
        c = lax.dot_general(ea_ref[...], w1c_ref[...],
                            (((0,), (0,)), ((), ())),
                            preferred_element_type=jnp.float32)
        m = m_ref[...].T
        pre = g_ref[...] + c + b1_ref[...]
        h = jnp.maximum(pre, 0.0).astype(jnp.bfloat16)
        ef_ref[...] = (jnp.dot(h, w2_ref[...],
                               preferred_element_type=jnp.float32)
                       + b2_ref[...]) * m

    return pl.pallas_call(
        body,
        grid=(nblk,),
        in_specs=[
            pl.BlockSpec((eb, d), lambda i: (i, 0)),
            pl.BlockSpec((de, eb), lambda i: (0, off + i)),
            pl.BlockSpec((1, eb), lambda i: (0, off + i)),
            pl.BlockSpec((de, d), lambda i: (0, 0)),
            pl.BlockSpec((1, d), lambda i: (0, 0)),
            pl.BlockSpec((d, d), lambda i: (0, 0)),
            pl.BlockSpec((1, d), lambda i: (0, 0)),
        ],
        out_specs=pl.BlockSpec((eb, d), lambda i: (i, 0)),
        out_shape=jax.ShapeDtypeStruct((es, d), jnp.float32),
    )(g, ea_p, mask_p, w1c, b1, w2_bf, b2)


def _sc_scatter_add(efs, row3d, n):
    ns = len(efs)
    es, d = efs[0].shape
    e = es * ns
    nw = NC * NS
    epw = es // nw
    ch = epw // C
    tr = 80
    nt = n // tr
    tpw = (nt + NS - 1) // NS

    mesh = plsc.VectorSubcoreMesh(core_axis_name="c", subcore_axis_name="s")

    def body(*refs):
        ef_hbms = refs[:ns]
        row_hbm = refs[ns]
        out_hbm = refs[ns + 1]
        acc, rowv, eb0, eb1, zbuf, se0, se1 = refs[ns + 2:]
        cid = lax.axis_index("c")
        sid = lax.axis_index("s")
        wid = sid * NC + cid

        def zrow(r, c2):
            for k in range(d // LANES):
                zbuf[r, pl.ds(k * LANES, LANES)] = jnp.zeros((LANES,), jnp.float32)
            return c2

        lax.fori_loop(0, tr, zrow, 0)
        for t in range(tpw):
            tile = sid + t * NS
            @pl.when(tile < nt)
            def _():
                pltpu.sync_copy(zbuf, acc.at[pl.ds(tile * tr, tr)])
        plsc.subcore_barrier()

        pltpu.sync_copy(row_hbm.at[wid], rowv)

        bufs = ((eb0, se0), (eb1, se1))

        def issue(si, jj, p):
            buf, sem = bufs[p]
            pltpu.async_copy(
                ef_hbms[si].at[pl.ds(wid * epw + jj * C, C)], buf, sem)

        issue(0, 0, 0)
        for si in range(ns):
            base_par = (si * ch) % 2

            def chunk(j, carry):
                for p in (0, 1):
                    @pl.when((base_par + j) % 2 == p)
                    def _():
                        buf, sem = bufs[p]

                        @pl.when(j + 1 < ch)
                        def _():
                            issue(si, j + 1, 1 - p)
                        if si + 1 < ns:
                            @pl.when(j + 1 >= ch)
                            def _():
                                issue(si + 1, 0, 1 - p)
                        pltpu.make_async_copy(
                            ef_hbms[si].at[pl.ds(wid * epw + j * C, C)],
                            buf, sem).wait()
                        pltpu.sync_copy(buf, acc.at[rowv.at[si * ch + j]],
                                        add=True)
                return carry

            lax.fori_loop(0, ch, chunk, 0)

        plsc.subcore_barrier()
        for t in range(tpw):
            tile = sid + t * NS
            @pl.when(tile < nt)
            def _():
                pltpu.sync_copy(acc.at[pl.ds(tile * tr, tr)],
                                out_hbm.at[cid, pl.ds(tile * tr, tr)])

    f = pl.kernel(
        body,
        out_type=jax.ShapeDtypeStruct((NC, n, d), jnp.float32),
        mesh=mesh,
        scratch_types=[
            pltpu.VMEM_SHARED((n, d), jnp.float32),
            pltpu.VMEM((ns * ch, C), jnp.int32),
            pltpu.VMEM((C, d), jnp.float32),
            pltpu.VMEM((C, d), jnp.float32),
            pltpu.VMEM((tr, d), jnp.float32),
            pltpu.SemaphoreType.DMA,
            pltpu.SemaphoreType.DMA,
        ],
    )
    return f(*efs, row3d)


def _node_mlp(x, aggp, wn1a, wn1b, bn1, wn2, bn2):
    n, d = x.shape
    nb = 1000

    def body(x_ref, a0_ref, a1_ref, wa_ref, wb_ref, b1_ref, w2_ref, b2_ref,
             out_ref):
        xv = x_ref[...]
        agg = a0_ref[0] + a1_ref[0]
        hn = jnp.maximum(
            jnp.dot(xv, wa_ref[...], preferred_element_type=jnp.float32)
            + jnp.dot(agg, wb_ref[...], preferred_element_type=jnp.float32)
            + b1_ref[...], 0.0)
        out_ref[...] = (xv
                        + jnp.dot(hn, w2_ref[...],
                                  preferred_element_type=jnp.float32)
                        + b2_ref[...])

    return pl.pallas_call(
        body,
        grid=(n // nb,),
        in_specs=[
            pl.BlockSpec((nb, d), lambda i: (i, 0)),
            pl.BlockSpec((1, nb, d), lambda i: (0, i, 0)),
            pl.BlockSpec((1, nb, d), lambda i: (1, i, 0)),
            pl.BlockSpec((d, d), lambda i: (0, 0)),
            pl.BlockSpec((d, d), lambda i: (0, 0)),
            pl.BlockSpec((1, d), lambda i: (0, 0)),
            pl.BlockSpec((d, d), lambda i: (0, 0)),
            pl.BlockSpec((1, d), lambda i: (0, 0)),
        ],
        out_specs=pl.BlockSpec((nb, d), lambda i: (i, 0)),
        out_shape=jax.ShapeDtypeStruct((n, d), jnp.float32),
    )(x, aggp, aggp, wn1a, wn1b, bn1, wn2, bn2)


def kernel(x, edge_index, edge_mask, edge_attr, W1, b1, W2, b2,
           Wn1, bn1, Wn2, bn2):
    n, d = x.shape
    e = edge_index.shape[1]
    nw = NC * NS
    ns = 5
    es = e // ns
    row4d = edge_index[0].reshape(ns, nw, es // (nw * C), C)
    col4d = edge_index[1].reshape(ns, nw, es // (nw * C), C)
    row_sc = row4d.transpose(1, 0, 2, 3).reshape(nw, e // (nw * C), C)

    pa, pb = _preproject(x, W1[:d], W1[d:2 * d])
    gs = [_sc_gather_add(pa, pb, row4d[i], col4d[i], es) for i in range(ns)]
    w1c = W1[2 * d:]
    w2_bf = W2.astype(jnp.bfloat16)
    b1r = b1.reshape(1, d)
    b2r = b2.reshape(1, d)
    ea_p = edge_attr.T
    mask_p = edge_mask.T
    efs = [_edge_tail_slice(gs[i], ea_p, mask_p, w1c,
                            b1r, w2_bf, b2r, i) for i in range(ns)]
    aggp = _sc_scatter_add(efs, row_sc, n)
    ef = jnp.concatenate(efs, axis=0)
    x_out = _node_mlp(x, aggp, Wn1[:d], Wn1[d:], bn1.reshape(1, d),
                      Wn2, bn2.reshape(1, d))
    return x_out, ef


# speedup vs baseline: 1.4809x; 1.1352x over previous
import functools

import jax
import jax.numpy as jnp
from jax import lax
from jax.experimental import pallas as pl
from jax.experimental.pallas import tpu as pltpu
from jax.experimental.pallas import tpu_sc as plsc

NC = 2
NS = 16
LANES = 16
C = 80


def _preproject(x, w1a, w1b):
    n, d = x.shape
    nb = 1000

    def body(x_ref, wa_ref, wb_ref, pa_ref, pb_ref):
        xv = x_ref[...]
        pa_ref[...] = jnp.dot(xv, wa_ref[...], preferred_element_type=jnp.float32)
        pb_ref[...] = jnp.dot(xv, wb_ref[...], preferred_element_type=jnp.float32)

    return pl.pallas_call(
        body,
        grid=(n // nb,),
        in_specs=[
            pl.BlockSpec((nb, d), lambda i: (i, 0)),
            pl.BlockSpec((d, d), lambda i: (0, 0)),
            pl.BlockSpec((d, d), lambda i: (0, 0)),
        ],
        out_specs=[pl.BlockSpec((nb, d), lambda i: (i, 0))] * 2,
        out_shape=[jax.ShapeDtypeStruct((n, d), jnp.float32)] * 2,
    )(x, w1a, w1b)


def _sc_gather_add(pa, pb, row3d, col3d, e):
    n, d = pa.shape
    nw = NC * NS
    epw = e // nw
    ch = epw // C

    mesh = plsc.VectorSubcoreMesh(core_axis_name="c", subcore_axis_name="s")

    def body(pa_hbm, pb_hbm, row_hbm, col_hbm, out_hbm,
             rowv, colv, ba0, bb0, ba1, bb1, sb0_, sb1_,
             sa0, sbm0, so0, sa1, sbm1, so1):
        cid = lax.axis_index("c")
        sid = lax.axis_index("s")
        wid = sid * NC + cid
        pltpu.sync_copy(row_hbm.at[wid], rowv)
        pltpu.sync_copy(col_hbm.at[wid], colv)

        bufs = ((ba0, bb0, sb0_, sa0, sbm0, so0),
                (ba1, bb1, sb1_, sa1, sbm1, so1))

        def out_slice(jj):
            return out_hbm.at[pl.ds(
                pl.multiple_of(wid * epw + jj * C, 16), C)]

        def issue(jj, p):
            ba, bb, _, sa, sb, _ = bufs[p]
            pltpu.async_copy(pa_hbm.at[rowv.at[jj]], ba, sa)
            pltpu.async_copy(pb_hbm.at[colv.at[jj]], bb, sb)

        def wait_store(jj, p):
            sbuf = bufs[p][2]
            so = bufs[p][5]
            pltpu.make_async_copy(sbuf, out_slice(jj), so).wait()

        issue(0, 0)

        def chunk(j, carry):
            for p in (0, 1):
                @pl.when(j % 2 == p)
                def _():
                    @pl.when(j + 1 < ch)
                    def _():
                        issue(j + 1, 1 - p)
                    pltpu.make_async_copy(
                        pa_hbm.at[rowv.at[j]], bufs[p][0], bufs[p][3]).wait()
                    pltpu.make_async_copy(
                        pb_hbm.at[colv.at[j]], bufs[p][1], bufs[p][4]).wait()
                    @pl.when(j >= 2)
                    def _():
                        wait_store(j - 2, p)

                    def addrow(r, c2):
                        sbuf = bufs[p][2]
                        for k in range(d // LANES):
                            s16 = pl.ds(k * LANES, LANES)
                            sbuf[r, s16] = bufs[p][0][r, s16] + bufs[p][1][r, s16]
                        return c2

                    lax.fori_loop(0, C, addrow, 0)
                    pltpu.async_copy(bufs[p][2], out_slice(j), bufs[p][5])
            return carry

        lax.fori_loop(0, ch, chunk, 0)
        wait_store(ch - 1, (ch - 1) % 2)
        wait_store(ch - 2, (ch - 2) % 2)

    f = pl.kernel(
        body,
        out_type=jax.ShapeDtypeStruct((e, d), jnp.float32),
        mesh=mesh,
        scratch_types=[
            pltpu.VMEM((ch, C), jnp.int32),
            pltpu.VMEM((ch, C), jnp.int32),
            pltpu.VMEM((C, d), jnp.float32),
            pltpu.VMEM((C, d), jnp.float32),
            pltpu.VMEM((C, d), jnp.float32),
            pltpu.VMEM((C, d), jnp.float32),
            pltpu.VMEM((C, d), jnp.float32),
            pltpu.VMEM((C, d), jnp.float32),
            pltpu.SemaphoreType.DMA,
            pltpu.SemaphoreType.DMA,
            pltpu.SemaphoreType.DMA,
            pltpu.SemaphoreType.DMA,
            pltpu.SemaphoreType.DMA,
            pltpu.SemaphoreType.DMA,
        ],
    )
    return f(pa, pb, row3d, col3d)


def _edge_tail_slice(g, ea_p, mask_p, w1c, b1, w2_bf, b2, ef_prev, slice_idx):
    es, d = g.shape
    e = ea_p.shape[1]
    de = 16
    eb = 1280
    nblk = es // eb
    off = slice_idx * nblk

    def body(*refs):
        g_ref, ea_ref, m_ref, w1c_ref, b1_ref, w2_ref, b2_ref = refs[:7]
        ef_ref = refs[-1]
        c = lax.dot_general(ea_ref[...], w1c_ref[...],
                            (((0,), (0,)), ((), ())),
                            preferred_element_type=jnp.float32)
        m = m_ref[...].T
        pre = g_ref[...] + c + b1_ref[...]
        h = jnp.maximum(pre, 0.0).astype(jnp.bfloat16)
        ef_ref[...] = (jnp.dot(h, w2_ref[...],
                               preferred_element_type=jnp.float32)
                       + b2_ref[...]) * m

    in_specs = [
        pl.BlockSpec((eb, d), lambda i: (i, 0)),
        pl.BlockSpec((de, eb), lambda i: (0, off + i)),
        pl.BlockSpec((1, eb), lambda i: (0, off + i)),
        pl.BlockSpec((de, d), lambda i: (0, 0)),
        pl.BlockSpec((1, d), lambda i: (0, 0)),
        pl.BlockSpec((d, d), lambda i: (0, 0)),
        pl.BlockSpec((1, d), lambda i: (0, 0)),
    ]
    args = [g, ea_p, mask_p, w1c, b1, w2_bf, b2]
    aliases = {}
    if ef_prev is not None:
        in_specs.append(pl.BlockSpec(memory_space=pl.ANY))
        args.append(ef_prev)
        aliases = {7: 0}

    return pl.pallas_call(
        body,
        grid=(nblk,),
        in_specs=in_specs,
        out_specs=pl.BlockSpec((eb, d), lambda i: (off + i, 0)),
        out_shape=jax.ShapeDtypeStruct((e, d), jnp.float32),
        input_output_aliases=aliases,
    )(*args)


def _sc_scatter_add(ef, row3d, n):
    e, d = ef.shape
    nw = NC * NS
    epw = e // nw
    ch = epw // C
    tr = 80
    nt = n // tr
    tpw = (nt + NS - 1) // NS

    mesh = plsc.VectorSubcoreMesh(core_axis_name="c", subcore_axis_name="s")

    def body(ef_hbm, row_hbm, out_hbm, acc, rowv, eb0, eb1, zbuf, se0, se1):
        cid = lax.axis_index("c")
        sid = lax.axis_index("s")
        wid = sid * NC + cid

        def zrow(r, c2):
            for k in range(d // LANES):
                zbuf[r, pl.ds(k * LANES, LANES)] = jnp.zeros((LANES,), jnp.float32)
            return c2

        lax.fori_loop(0, tr, zrow, 0)
        for t in range(tpw):
            tile = sid + t * NS
            @pl.when(tile < nt)
            def _():
                pltpu.sync_copy(zbuf, acc.at[pl.ds(tile * tr, tr)])
        plsc.subcore_barrier()

        pltpu.sync_copy(row_hbm.at[wid], rowv)

        bufs = ((eb0, se0), (eb1, se1))

        def issue(jj, p):
            buf, sem = bufs[p]
            pltpu.async_copy(ef_hbm.at[pl.ds(wid * epw + jj * C, C)], buf, sem)

        issue(0, 0)

        def chunk(j, carry):
            for p in (0, 1):
                @pl.when(j % 2 == p)
                def _():
                    buf, sem = bufs[p]

                    @pl.when(j + 1 < ch)
                    def _():
                        issue(j + 1, 1 - p)
                    pltpu.make_async_copy(
                        ef_hbm.at[pl.ds(wid * epw + j * C, C)], buf, sem).wait()
                    pltpu.sync_copy(buf, acc.at[rowv.at[j]], add=True)
            return carry

        lax.fori_loop(0, ch, chunk, 0)
        plsc.subcore_barrier()
        for t in range(tpw):
            tile = sid + t * NS
            @pl.when(tile < nt)
            def _():
                pltpu.sync_copy(acc.at[pl.ds(tile * tr, tr)],
                                out_hbm.at[cid, pl.ds(tile * tr, tr)])

    f = pl.kernel(
        body,
        out_type=jax.ShapeDtypeStruct((NC, n, d), jnp.float32),
        mesh=mesh,
        scratch_types=[
            pltpu.VMEM_SHARED((n, d), jnp.float32),
            pltpu.VMEM((ch, C), jnp.int32),
            pltpu.VMEM((C, d), jnp.float32),
            pltpu.VMEM((C, d), jnp.float32),
            pltpu.VMEM((tr, d), jnp.float32),
            pltpu.SemaphoreType.DMA,
            pltpu.SemaphoreType.DMA,
        ],
    )
    return f(ef, row3d)


def _node_mlp(x, aggp, wn1a, wn1b, bn1, wn2, bn2):
    n, d = x.shape
    nb = 1000

    def body(x_ref, a0_ref, a1_ref, wa_ref, wb_ref, b1_ref, w2_ref, b2_ref,
             out_ref):
        xv = x_ref[...]
        agg = a0_ref[0] + a1_ref[0]
        hn = jnp.maximum(
            jnp.dot(xv, wa_ref[...], preferred_element_type=jnp.float32)
            + jnp.dot(agg, wb_ref[...], preferred_element_type=jnp.float32)
            + b1_ref[...], 0.0)
        out_ref[...] = (xv
                        + jnp.dot(hn, w2_ref[...],
                                  preferred_element_type=jnp.float32)
                        + b2_ref[...])

    return pl.pallas_call(
        body,
        grid=(n // nb,),
        in_specs=[
            pl.BlockSpec((nb, d), lambda i: (i, 0)),
            pl.BlockSpec((1, nb, d), lambda i: (0, i, 0)),
            pl.BlockSpec((1, nb, d), lambda i: (1, i, 0)),
            pl.BlockSpec((d, d), lambda i: (0, 0)),
            pl.BlockSpec((d, d), lambda i: (0, 0)),
            pl.BlockSpec((1, d), lambda i: (0, 0)),
            pl.BlockSpec((d, d), lambda i: (0, 0)),
            pl.BlockSpec((1, d), lambda i: (0, 0)),
        ],
        out_specs=pl.BlockSpec((nb, d), lambda i: (i, 0)),
        out_shape=jax.ShapeDtypeStruct((n, d), jnp.float32),
    )(x, aggp, aggp, wn1a, wn1b, bn1, wn2, bn2)


def kernel(x, edge_index, edge_mask, edge_attr, W1, b1, W2, b2,
           Wn1, bn1, Wn2, bn2):
    n, d = x.shape
    e = edge_index.shape[1]
    nw = NC * NS
    ns = 5
    es = e // ns
    row4d = edge_index[0].reshape(ns, nw, es // (nw * C), C)
    col4d = edge_index[1].reshape(ns, nw, es // (nw * C), C)
    row3d = edge_index[0].reshape(nw, e // (nw * C), C)

    pa, pb = _preproject(x, W1[:d], W1[d:2 * d])
    gs = [_sc_gather_add(pa, pb, row4d[i], col4d[i], es) for i in range(ns)]
    w1c = W1[2 * d:]
    w2_bf = W2.astype(jnp.bfloat16)
    b1r = b1.reshape(1, d)
    b2r = b2.reshape(1, d)
    ea_p = edge_attr.T
    mask_p = edge_mask.T
    ef = None
    for i in range(ns):
        ef = _edge_tail_slice(gs[i], ea_p, mask_p, w1c,
                              b1r, w2_bf, b2r, ef, i)
    aggp = _sc_scatter_add(ef, row3d, n)
    x_out = _node_mlp(x, aggp, Wn1[:d], Wn1[d:], bn1.reshape(1, d),
                      Wn2, bn2.reshape(1, d))
    return x_out, ef


# speedup vs baseline: 1.6424x; 1.1091x over previous
import functools

import jax
import jax.numpy as jnp
from jax import lax
from jax.experimental import pallas as pl
from jax.experimental.pallas import tpu as pltpu
from jax.experimental.pallas import tpu_sc as plsc

NC = 2
NS = 16
LANES = 16
C = 80


def _preproject(x, w1a, w1b):
    n, d = x.shape
    nb = 1000

    def body(x_ref, wa_ref, wb_ref, pa_ref, pb_ref):
        xv = x_ref[...]
        pa_ref[...] = jnp.dot(xv, wa_ref[...], preferred_element_type=jnp.float32)
        pb_ref[...] = jnp.dot(xv, wb_ref[...], preferred_element_type=jnp.float32)

    return pl.pallas_call(
        body,
        grid=(n // nb,),
        in_specs=[
            pl.BlockSpec((nb, d), lambda i: (i, 0)),
            pl.BlockSpec((d, d), lambda i: (0, 0)),
            pl.BlockSpec((d, d), lambda i: (0, 0)),
        ],
        out_specs=[pl.BlockSpec((nb, d), lambda i: (i, 0))] * 2,
        out_shape=[jax.ShapeDtypeStruct((n, d), jnp.float32)] * 2,
    )(x, w1a, w1b)


def _sc_gather_add(pa, pb, row3d, col3d, e):
    n, d = pa.shape
    nw = NC * NS
    epw = e // nw
    ch = epw // C

    mesh = plsc.VectorSubcoreMesh(core_axis_name="c", subcore_axis_name="s")

    def body(pa_hbm, pb_hbm, row_hbm, col_hbm, out_hbm,
             rowv, colv, ba0, bb0, ba1, bb1, sb0_, sb1_,
             sa0, sbm0, so0, sa1, sbm1, so1):
        cid = lax.axis_index("c")
        sid = lax.axis_index("s")
        wid = sid * NC + cid
        pltpu.sync_copy(row_hbm.at[wid], rowv)
        pltpu.sync_copy(col_hbm.at[wid], colv)

        bufs = ((ba0, bb0, sb0_, sa0, sbm0, so0),
                (ba1, bb1, sb1_, sa1, sbm1, so1))

        def out_slice(jj):
            return out_hbm.at[pl.ds(
                pl.multiple_of(wid * epw + jj * C, 16), C)]

        def issue(jj, p):
            ba, bb, _, sa, sb, _ = bufs[p]
            pltpu.async_copy(pa_hbm.at[rowv.at[jj]], ba, sa)
            pltpu.async_copy(pb_hbm.at[colv.at[jj]], bb, sb)

        def wait_store(jj, p):
            sbuf = bufs[p][2]
            so = bufs[p][5]
            pltpu.make_async_copy(sbuf, out_slice(jj), so).wait()

        issue(0, 0)

        def chunk(j, carry):
            for p in (0, 1):
                @pl.when(j % 2 == p)
                def _():
                    @pl.when(j + 1 < ch)
                    def _():
                        issue(j + 1, 1 - p)
                    pltpu.make_async_copy(
                        pa_hbm.at[rowv.at[j]], bufs[p][0], bufs[p][3]).wait()
                    pltpu.make_async_copy(
                        pb_hbm.at[colv.at[j]], bufs[p][1], bufs[p][4]).wait()
                    @pl.when(j >= 2)
                    def _():
                        wait_store(j - 2, p)

                    def addrow(r, c2):
                        sbuf = bufs[p][2]
                        for k in range(d // LANES):
                            s16 = pl.ds(k * LANES, LANES)
                            sbuf[r, s16] = bufs[p][0][r, s16] + bufs[p][1][r, s16]
                        return c2

                    lax.fori_loop(0, C, addrow, 0)
                    pltpu.async_copy(bufs[p][2], out_slice(j), bufs[p][5])
            return carry

        lax.fori_loop(0, ch, chunk, 0)
        wait_store(ch - 1, (ch - 1) % 2)
        wait_store(ch - 2, (ch - 2) % 2)

    f = pl.kernel(
        body,
        out_type=jax.ShapeDtypeStruct((e, d), jnp.float32),
        mesh=mesh,
        scratch_types=[
            pltpu.VMEM((ch, C), jnp.int32),
            pltpu.VMEM((ch, C), jnp.int32),
            pltpu.VMEM((C, d), jnp.float32),
            pltpu.VMEM((C, d), jnp.float32),
            pltpu.VMEM((C, d), jnp.float32),
            pltpu.VMEM((C, d), jnp.float32),
            pltpu.VMEM((C, d), jnp.float32),
            pltpu.VMEM((C, d), jnp.float32),
            pltpu.SemaphoreType.DMA,
            pltpu.SemaphoreType.DMA,
            pltpu.SemaphoreType.DMA,
            pltpu.SemaphoreType.DMA,
            pltpu.SemaphoreType.DMA,
            pltpu.SemaphoreType.DMA,
        ],
    )
    return f(pa, pb, row3d, col3d)


def _edge_tail_slice(g, ea_p, mask_p, w1c, b1, w2_bf, b2, ef_prev, slice_idx):
    es, d = g.shape
    e = ea_p.shape[1]
    de = 16
    eb = 3200
    nblk = es // eb
    off = slice_idx * nblk

    def body(*refs):
        g_ref, ea_ref, m_ref, w1c_ref, b1_ref, w2_ref, b2_ref = refs[:7]
        ef_ref = refs[-1]
        c = lax.dot_general(ea_ref[...], w1c_ref[...],
                            (((0,), (0,)), ((), ())),
                            preferred_element_type=jnp.float32)
        m = m_ref[...].T
        pre = g_ref[...] + c + b1_ref[...]
        h = jnp.maximum(pre, 0.0).astype(jnp.bfloat16)
        ef_ref[...] = (jnp.dot(h, w2_ref[...],
                               preferred_element_type=jnp.float32)
                       + b2_ref[...]) * m

    in_specs = [
        pl.BlockSpec((eb, d), lambda i: (i, 0)),
        pl.BlockSpec((de, eb), lambda i: (0, off + i)),
        pl.BlockSpec((1, eb), lambda i: (0, off + i)),
        pl.BlockSpec((de, d), lambda i: (0, 0)),
        pl.BlockSpec((1, d), lambda i: (0, 0)),
        pl.BlockSpec((d, d), lambda i: (0, 0)),
        pl.BlockSpec((1, d), lambda i: (0, 0)),
    ]
    args = [g, ea_p, mask_p, w1c, b1, w2_bf, b2]
    aliases = {}
    if ef_prev is not None:
        in_specs.append(pl.BlockSpec(memory_space=pl.ANY))
        args.append(ef_prev)
        aliases = {7: 0}

    return pl.pallas_call(
        body,
        grid=(nblk,),
        in_specs=in_specs,
        out_specs=pl.BlockSpec((eb, d), lambda i: (off + i, 0)),
        out_shape=jax.ShapeDtypeStruct((e, d), jnp.float32),
        input_output_aliases=aliases,
    )(*args)


def _sc_scatter_add(ef, row3d, n):
    e, d = ef.shape
    nw = NC * NS
    epw = e // nw
    ch = epw // C
    tr = 80
    nt = n // tr
    tpw = (nt + NS - 1) // NS

    mesh = plsc.VectorSubcoreMesh(core_axis_name="c", subcore_axis_name="s")

    def body(ef_hbm, row_hbm, out_hbm, acc, rowv, eb0, eb1, zbuf, se0, se1):
        cid = lax.axis_index("c")
        sid = lax.axis_index("s")
        wid = sid * NC + cid

        def zrow(r, c2):
            for k in range(d // LANES):
                zbuf[r, pl.ds(k * LANES, LANES)] = jnp.zeros((LANES,), jnp.float32)
            return c2

        lax.fori_loop(0, tr, zrow, 0)
        for t in range(tpw):
            tile = sid + t * NS
            @pl.when(tile < nt)
            def _():
                pltpu.sync_copy(zbuf, acc.at[pl.ds(tile * tr, tr)])
        plsc.subcore_barrier()

        pltpu.sync_copy(row_hbm.at[wid], rowv)

        bufs = ((eb0, se0), (eb1, se1))

        def issue(jj, p):
            buf, sem = bufs[p]
            pltpu.async_copy(ef_hbm.at[pl.ds(wid * epw + jj * C, C)], buf, sem)

        issue(0, 0)

        def chunk(j, carry):
            for p in (0, 1):
                @pl.when(j % 2 == p)
                def _():
                    buf, sem = bufs[p]

                    @pl.when(j + 1 < ch)
                    def _():
                        issue(j + 1, 1 - p)
                    pltpu.make_async_copy(
                        ef_hbm.at[pl.ds(wid * epw + j * C, C)], buf, sem).wait()
                    pltpu.sync_copy(buf, acc.at[rowv.at[j]], add=True)
            return carry

        lax.fori_loop(0, ch, chunk, 0)
        plsc.subcore_barrier()
        for t in range(tpw):
            tile = sid + t * NS
            @pl.when(tile < nt)
            def _():
                pltpu.sync_copy(acc.at[pl.ds(tile * tr, tr)],
                                out_hbm.at[cid, pl.ds(tile * tr, tr)])

    f = pl.kernel(
        body,
        out_type=jax.ShapeDtypeStruct((NC, n, d), jnp.float32),
        mesh=mesh,
        scratch_types=[
            pltpu.VMEM_SHARED((n, d), jnp.float32),
            pltpu.VMEM((ch, C), jnp.int32),
            pltpu.VMEM((C, d), jnp.float32),
            pltpu.VMEM((C, d), jnp.float32),
            pltpu.VMEM((tr, d), jnp.float32),
            pltpu.SemaphoreType.DMA,
            pltpu.SemaphoreType.DMA,
        ],
    )
    return f(ef, row3d)


def _node_mlp(x, aggp, wn1a, wn1b, bn1, wn2, bn2):
    n, d = x.shape
    nb = 1000

    def body(x_ref, a0_ref, a1_ref, wa_ref, wb_ref, b1_ref, w2_ref, b2_ref,
             out_ref):
        xv = x_ref[...]
        agg = a0_ref[0] + a1_ref[0]
        hn = jnp.maximum(
            jnp.dot(xv, wa_ref[...], preferred_element_type=jnp.float32)
            + jnp.dot(agg, wb_ref[...], preferred_element_type=jnp.float32)
            + b1_ref[...], 0.0)
        out_ref[...] = (xv
                        + jnp.dot(hn, w2_ref[...],
                                  preferred_element_type=jnp.float32)
                        + b2_ref[...])

    return pl.pallas_call(
        body,
        grid=(n // nb,),
        in_specs=[
            pl.BlockSpec((nb, d), lambda i: (i, 0)),
            pl.BlockSpec((1, nb, d), lambda i: (0, i, 0)),
            pl.BlockSpec((1, nb, d), lambda i: (1, i, 0)),
            pl.BlockSpec((d, d), lambda i: (0, 0)),
            pl.BlockSpec((d, d), lambda i: (0, 0)),
            pl.BlockSpec((1, d), lambda i: (0, 0)),
            pl.BlockSpec((d, d), lambda i: (0, 0)),
            pl.BlockSpec((1, d), lambda i: (0, 0)),
        ],
        out_specs=pl.BlockSpec((nb, d), lambda i: (i, 0)),
        out_shape=jax.ShapeDtypeStruct((n, d), jnp.float32),
    )(x, aggp, aggp, wn1a, wn1b, bn1, wn2, bn2)


def kernel(x, edge_index, edge_mask, edge_attr, W1, b1, W2, b2,
           Wn1, bn1, Wn2, bn2):
    n, d = x.shape
    e = edge_index.shape[1]
    nw = NC * NS
    ns = 5
    es = e // ns
    row4d = edge_index[0].reshape(ns, nw, es // (nw * C), C)
    col4d = edge_index[1].reshape(ns, nw, es // (nw * C), C)
    row3d = edge_index[0].reshape(nw, e // (nw * C), C)

    pa, pb = _preproject(x, W1[:d], W1[d:2 * d])
    gs = [_sc_gather_add(pa, pb, row4d[i], col4d[i], es) for i in range(ns)]
    w1c = W1[2 * d:]
    w2_bf = W2.astype(jnp.bfloat16)
    b1r = b1.reshape(1, d)
    b2r = b2.reshape(1, d)
    ea_p = edge_attr.T
    mask_p = edge_mask.T
    ef = None
    for i in range(ns):
        ef = _edge_tail_slice(gs[i], ea_p, mask_p, w1c,
                              b1r, w2_bf, b2r, ef, i)
    aggp = _sc_scatter_add(ef, row3d, n)
    x_out = _node_mlp(x, aggp, Wn1[:d], Wn1[d:], bn1.reshape(1, d),
                      Wn2, bn2.reshape(1, d))
    return x_out, ef


# speedup vs baseline: 1.6563x; 1.0085x over previous
import functools

import jax
import jax.numpy as jnp
from jax import lax
from jax.experimental import pallas as pl
from jax.experimental.pallas import tpu as pltpu
from jax.experimental.pallas import tpu_sc as plsc

NC = 2
NS = 16
LANES = 16
C = 80


def _preproject(x, w1a, w1b):
    n, d = x.shape
    nb = 1000

    def body(x_ref, wa_ref, wb_ref, pa_ref, pb_ref):
        xv = x_ref[...]
        pa_ref[...] = jnp.dot(xv, wa_ref[...], preferred_element_type=jnp.float32)
        pb_ref[...] = jnp.dot(xv, wb_ref[...], preferred_element_type=jnp.float32)

    return pl.pallas_call(
        body,
        grid=(n // nb,),
        in_specs=[
            pl.BlockSpec((nb, d), lambda i: (i, 0)),
            pl.BlockSpec((d, d), lambda i: (0, 0)),
            pl.BlockSpec((d, d), lambda i: (0, 0)),
        ],
        out_specs=[pl.BlockSpec((nb, d), lambda i: (i, 0))] * 2,
        out_shape=[jax.ShapeDtypeStruct((n, d), jnp.float32)] * 2,
    )(x, w1a, w1b)


def _sc_gather_add(pa, pb, row3d, col3d, e):
    n, d = pa.shape
    nw = NC * NS
    epw = e // nw
    ch = epw // C

    mesh = plsc.VectorSubcoreMesh(core_axis_name="c", subcore_axis_name="s")

    def body(pa_hbm, pb_hbm, row_hbm, col_hbm, out_hbm,
             rowv, colv, ba0, bb0, ba1, bb1, sb0_, sb1_,
             sa0, sbm0, so0, sa1, sbm1, so1):
        cid = lax.axis_index("c")
        sid = lax.axis_index("s")
        wid = sid * NC + cid
        pltpu.sync_copy(row_hbm.at[wid], rowv)
        pltpu.sync_copy(col_hbm.at[wid], colv)

        bufs = ((ba0, bb0, sb0_, sa0, sbm0, so0),
                (ba1, bb1, sb1_, sa1, sbm1, so1))

        def out_slice(jj):
            return out_hbm.at[pl.ds(
                pl.multiple_of(wid * epw + jj * C, 16), C)]

        def issue(jj, p):
            ba, bb, _, sa, sb, _ = bufs[p]
            pltpu.async_copy(pa_hbm.at[rowv.at[jj]], ba, sa)
            pltpu.async_copy(pb_hbm.at[colv.at[jj]], bb, sb)

        def wait_store(jj, p):
            sbuf = bufs[p][2]
            so = bufs[p][5]
            pltpu.make_async_copy(sbuf, out_slice(jj), so).wait()

        issue(0, 0)

        def chunk(j, carry):
            for p in (0, 1):
                @pl.when(j % 2 == p)
                def _():
                    @pl.when(j + 1 < ch)
                    def _():
                        issue(j + 1, 1 - p)
                    pltpu.make_async_copy(
                        pa_hbm.at[rowv.at[j]], bufs[p][0], bufs[p][3]).wait()
                    pltpu.make_async_copy(
                        pb_hbm.at[colv.at[j]], bufs[p][1], bufs[p][4]).wait()
                    @pl.when(j >= 2)
                    def _():
                        wait_store(j - 2, p)

                    def addrow(r, c2):
                        sbuf = bufs[p][2]
                        for k in range(d // LANES):
                            s16 = pl.ds(k * LANES, LANES)
                            sbuf[r, s16] = bufs[p][0][r, s16] + bufs[p][1][r, s16]
                        return c2

                    lax.fori_loop(0, C, addrow, 0)
                    pltpu.async_copy(bufs[p][2], out_slice(j), bufs[p][5])
            return carry

        lax.fori_loop(0, ch, chunk, 0)
        wait_store(ch - 1, (ch - 1) % 2)
        wait_store(ch - 2, (ch - 2) % 2)

    f = pl.kernel(
        body,
        out_type=jax.ShapeDtypeStruct((e, d), jnp.float32),
        mesh=mesh,
        scratch_types=[
            pltpu.VMEM((ch, C), jnp.int32),
            pltpu.VMEM((ch, C), jnp.int32),
            pltpu.VMEM((C, d), jnp.float32),
            pltpu.VMEM((C, d), jnp.float32),
            pltpu.VMEM((C, d), jnp.float32),
            pltpu.VMEM((C, d), jnp.float32),
            pltpu.VMEM((C, d), jnp.float32),
            pltpu.VMEM((C, d), jnp.float32),
            pltpu.SemaphoreType.DMA,
            pltpu.SemaphoreType.DMA,
            pltpu.SemaphoreType.DMA,
            pltpu.SemaphoreType.DMA,
            pltpu.SemaphoreType.DMA,
            pltpu.SemaphoreType.DMA,
        ],
    )
    return f(pa, pb, row3d, col3d)


def _edge_tail_slice(g, ea_p, mask_p, w1c, b1, w2_bf, b2, ef_prev, slice_idx):
    es, d = g.shape
    e = ea_p.shape[1]
    de = 16
    eb = 6400
    nblk = es // eb
    off = slice_idx * nblk

    def body(*refs):
        g_ref, ea_ref, m_ref, w1c_ref, b1_ref, w2_ref, b2_ref = refs[:7]
        ef_ref = refs[-1]
        c = lax.dot_general(ea_ref[...], w1c_ref[...],
                            (((0,), (0,)), ((), ())),
                            preferred_element_type=jnp.float32)
        m = m_ref[...].T
        pre = g_ref[...] + c + b1_ref[...]
        h = jnp.maximum(pre, 0.0).astype(jnp.bfloat16)
        ef_ref[...] = (jnp.dot(h, w2_ref[...],
                               preferred_element_type=jnp.float32)
                       + b2_ref[...]) * m

    in_specs = [
        pl.BlockSpec((eb, d), lambda i: (i, 0)),
        pl.BlockSpec((de, eb), lambda i: (0, off + i)),
        pl.BlockSpec((1, eb), lambda i: (0, off + i)),
        pl.BlockSpec((de, d), lambda i: (0, 0)),
        pl.BlockSpec((1, d), lambda i: (0, 0)),
        pl.BlockSpec((d, d), lambda i: (0, 0)),
        pl.BlockSpec((1, d), lambda i: (0, 0)),
    ]
    args = [g, ea_p, mask_p, w1c, b1, w2_bf, b2]
    aliases = {}
    if ef_prev is not None:
        in_specs.append(pl.BlockSpec(memory_space=pl.ANY))
        args.append(ef_prev)
        aliases = {7: 0}

    return pl.pallas_call(
        body,
        grid=(nblk,),
        in_specs=in_specs,
        out_specs=pl.BlockSpec((eb, d), lambda i: (off + i, 0)),
        out_shape=jax.ShapeDtypeStruct((e, d), jnp.float32),
        input_output_aliases=aliases,
    )(*args)


def _sc_scatter_add(ef, row3d, n):
    e, d = ef.shape
    nw = NC * NS
    epw = e // nw
    ch = epw // C
    tr = 80
    nt = n // tr
    tpw = (nt + NS - 1) // NS

    mesh = plsc.VectorSubcoreMesh(core_axis_name="c", subcore_axis_name="s")

    def body(ef_hbm, row_hbm, out_hbm, acc, rowv, eb0, eb1, zbuf, se0, se1):
        cid = lax.axis_index("c")
        sid = lax.axis_index("s")
        wid = sid * NC + cid

        def zrow(r, c2):
            for k in range(d // LANES):
                zbuf[r, pl.ds(k * LANES, LANES)] = jnp.zeros((LANES,), jnp.float32)
            return c2

        lax.fori_loop(0, tr, zrow, 0)
        for t in range(tpw):
            tile = sid + t * NS
            @pl.when(tile < nt)
            def _():
                pltpu.sync_copy(zbuf, acc.at[pl.ds(tile * tr, tr)])
        plsc.subcore_barrier()

        pltpu.sync_copy(row_hbm.at[wid], rowv)

        bufs = ((eb0, se0), (eb1, se1))

        def issue(jj, p):
            buf, sem = bufs[p]
            pltpu.async_copy(ef_hbm.at[pl.ds(wid * epw + jj * C, C)], buf, sem)

        issue(0, 0)

        def chunk(j, carry):
            for p in (0, 1):
                @pl.when(j % 2 == p)
                def _():
                    buf, sem = bufs[p]

                    @pl.when(j + 1 < ch)
                    def _():
                        issue(j + 1, 1 - p)
                    pltpu.make_async_copy(
                        ef_hbm.at[pl.ds(wid * epw + j * C, C)], buf, sem).wait()
                    pltpu.sync_copy(buf, acc.at[rowv.at[j]], add=True)
            return carry

        lax.fori_loop(0, ch, chunk, 0)
        plsc.subcore_barrier()
        for t in range(tpw):
            tile = sid + t * NS
            @pl.when(tile < nt)
            def _():
                pltpu.sync_copy(acc.at[pl.ds(tile * tr, tr)],
                                out_hbm.at[cid, pl.ds(tile * tr, tr)])

    f = pl.kernel(
        body,
        out_type=jax.ShapeDtypeStruct((NC, n, d), jnp.float32),
        mesh=mesh,
        scratch_types=[
            pltpu.VMEM_SHARED((n, d), jnp.float32),
            pltpu.VMEM((ch, C), jnp.int32),
            pltpu.VMEM((C, d), jnp.float32),
            pltpu.VMEM((C, d), jnp.float32),
            pltpu.VMEM((tr, d), jnp.float32),
            pltpu.SemaphoreType.DMA,
            pltpu.SemaphoreType.DMA,
        ],
    )
    return f(ef, row3d)


def _node_mlp(x, aggp, wn1a, wn1b, bn1, wn2, bn2):
    n, d = x.shape
    nb = 1000

    def body(x_ref, a0_ref, a1_ref, wa_ref, wb_ref, b1_ref, w2_ref, b2_ref,
             out_ref):
        xv = x_ref[...]
        agg = a0_ref[0] + a1_ref[0]
        hn = jnp.maximum(
            jnp.dot(xv, wa_ref[...], preferred_element_type=jnp.float32)
            + jnp.dot(agg, wb_ref[...], preferred_element_type=jnp.float32)
            + b1_ref[...], 0.0)
        out_ref[...] = (xv
                        + jnp.dot(hn, w2_ref[...],
                                  preferred_element_type=jnp.float32)
                        + b2_ref[...])

    return pl.pallas_call(
        body,
        grid=(n // nb,),
        in_specs=[
            pl.BlockSpec((nb, d), lambda i: (i, 0)),
            pl.BlockSpec((1, nb, d), lambda i: (0, i, 0)),
            pl.BlockSpec((1, nb, d), lambda i: (1, i, 0)),
            pl.BlockSpec((d, d), lambda i: (0, 0)),
            pl.BlockSpec((d, d), lambda i: (0, 0)),
            pl.BlockSpec((1, d), lambda i: (0, 0)),
            pl.BlockSpec((d, d), lambda i: (0, 0)),
            pl.BlockSpec((1, d), lambda i: (0, 0)),
        ],
        out_specs=pl.BlockSpec((nb, d), lambda i: (i, 0)),
        out_shape=jax.ShapeDtypeStruct((n, d), jnp.float32),
    )(x, aggp, aggp, wn1a, wn1b, bn1, wn2, bn2)


def kernel(x, edge_index, edge_mask, edge_attr, W1, b1, W2, b2,
           Wn1, bn1, Wn2, bn2):
    n, d = x.shape
    e = edge_index.shape[1]
    nw = NC * NS
    ns = 5
    es = e // ns
    row4d = edge_index[0].reshape(ns, nw, es // (nw * C), C)
    col4d = edge_index[1].reshape(ns, nw, es // (nw * C), C)
    row3d = edge_index[0].reshape(nw, e // (nw * C), C)

    pa, pb = _preproject(x, W1[:d], W1[d:2 * d])
    gs = [_sc_gather_add(pa, pb, row4d[i], col4d[i], es) for i in range(ns)]
    w1c = W1[2 * d:]
    w2_bf = W2.astype(jnp.bfloat16)
    b1r = b1.reshape(1, d)
    b2r = b2.reshape(1, d)
    ea_p = edge_attr.T
    mask_p = edge_mask.T
    ef = None
    for i in range(ns):
        ef = _edge_tail_slice(gs[i], ea_p, mask_p, w1c,
                              b1r, w2_bf, b2r, ef, i)
    aggp = _sc_scatter_add(ef, row3d, n)
    x_out = _node_mlp(x, aggp, Wn1[:d], Wn1[d:], bn1.reshape(1, d),
                      Wn2, bn2.reshape(1, d))
    return x_out, ef
